# baseline (speedup 1.0000x reference)
import functools

import jax
import jax.numpy as jnp
from jax import lax
from jax.experimental import pallas as pl
from jax.experimental.pallas import tpu as pltpu
from jax.experimental.pallas import tpu_sc as plsc

L1 = 3072
NPSQT = 8
D = L1 + NPSQT
H = L1 // 2
B = 1024
A = 32
NC = 2
NS = 16
NW = NC * NS
SPW = B // NW
CH = L1 // 16
ROWS_PER_GATHER = 16
ACC = L1 + 16


def _sc_kernel(w_idx, w_val, b_idx, b_val, us, them, weight, bias, ftv, out,
               idxw_v, idxb_v, vw_v, vb_v, us_v, them_v, ft_v, bias_v,
               rows0, rows1, acc_w, acc_b, out_stage, sem0, sem1):
    wid = lax.axis_index("s") * NC + lax.axis_index("c")
    base = wid * SPW

    pltpu.sync_copy(w_idx.at[pl.ds(base, SPW)], idxw_v)
    pltpu.sync_copy(b_idx.at[pl.ds(base, SPW)], idxb_v)
    pltpu.sync_copy(w_val.at[pl.ds(base, SPW)], vw_v)
    pltpu.sync_copy(b_val.at[pl.ds(base, SPW)], vb_v)
    pltpu.sync_copy(us.at[pl.ds(base, SPW)], us_v)
    pltpu.sync_copy(them.at[pl.ds(base, SPW)], them_v)
    pltpu.sync_copy(bias, bias_v)
    pltpu.sync_copy(ftv, ft_v)

    lane = lax.broadcasted_iota(jnp.int32, (16,), 0)
    tail_mask = (lane >= 8).astype(jnp.float32)
    rows = (rows0, rows1)
    sems = (sem0, sem1)
    idxs = (idxw_v, idxb_v)
    vals = (vw_v, vb_v)

    def start_gather(s, p, h):
        src = weight.at[idxs[p].at[s, pl.ds(h * ROWS_PER_GATHER, ROWS_PER_GATHER)]]
        return pltpu.async_copy(src, rows[h], sems[h])

    def full16(v):
        return jnp.full((16,), v, dtype=jnp.int32)

    def mac(s, p, h):
        acc = (acc_w, acc_b)[p]
        rbuf = rows[h]
        vref = vals[p]

        @pl.loop(0, ROWS_PER_GATHER)
        def _row(j):
            vj = plsc.load_gather(vref, [full16(s), full16(h * ROWS_PER_GATHER + j)])

            @pl.loop(0, CH, unroll=8)
            def _chunk(c):
                off = c * 16
                plsc.addupdate(acc.at[pl.ds(off, 16)], rbuf[j, pl.ds(off, 16)] * vj)

            tail = plsc.load_gather(rbuf, [full16(j), lane + (D - 16)])
            plsc.addupdate(acc.at[pl.ds(L1, 16)], tail * (vj * tail_mask))

    start_gather(0, 0, 0)
    start_gather(0, 0, 1)

    @pl.loop(0, SPW)
    def _sample(s):
        g = base + s
        s_next = jnp.minimum(s + 1, SPW - 1)

        @pl.loop(0, CH)
        def _init(c):
            off = c * 16
            bc = bias_v[pl.ds(off, 16)]
            acc_w[pl.ds(off, 16)] = bc
            acc_b[pl.ds(off, 16)] = bc
        btail = bias_v[pl.ds(D - 16, 16)]
        acc_w[pl.ds(L1, 16)] = btail
        acc_b[pl.ds(L1, 16)] = btail

        pltpu.make_async_copy(
            weight.at[idxw_v.at[s, pl.ds(0, ROWS_PER_GATHER)]], rows0, sem0).wait()
        mac(s, 0, 0)
        d_b0 = start_gather(s, 1, 0)

        pltpu.make_async_copy(
            weight.at[idxw_v.at[s, pl.ds(ROWS_PER_GATHER, ROWS_PER_GATHER)]], rows1, sem1).wait()
        mac(s, 0, 1)
        d_b1 = start_gather(s, 1, 1)

        d_b0.wait()
        mac(s, 1, 0)
        start_gather(s_next, 0, 0)

        d_b1.wait()
        mac(s, 1, 1)
        start_gather(s_next, 0, 1)

        usv = plsc.load_gather(us_v, [full16(s)])
        thv = plsc.load_gather(them_v, [full16(s)])
        ftm = ft_v[...]
        inv = 1.0 / ftm
        zero = jnp.zeros((16,), jnp.float32)

        def clampf(x):
            return jnp.minimum(jnp.maximum(x, zero), ftm)

        @pl.loop(0, H // 16, unroll=2)
        def _fin(c):
            o1 = c * 16
            o2 = H + c * 16
            aw1 = acc_w[pl.ds(o1, 16)]
            aw2 = acc_w[pl.ds(o2, 16)]
            ab1 = acc_b[pl.ds(o1, 16)]
            ab2 = acc_b[pl.ds(o2, 16)]
            w1 = clampf(usv * aw1 + thv * ab1)
            w2 = clampf(usv * aw2 + thv * ab2)
            v1 = clampf(usv * ab1 + thv * aw1)
            v2 = clampf(usv * ab2 + thv * aw2)
            out_stage[pl.ds(o1, 16)] = w1 * w2 * inv
            out_stage[pl.ds(H + o1, 16)] = v1 * v2 * inv

        pq = (acc_w[pl.ds(L1, 16)] - acc_b[pl.ds(L1, 16)]) * (usv - 0.5)
        plsc.store_scatter(out_stage, [lane + (D - 16)], pq, mask=lane >= 8)

        pltpu.sync_copy(out_stage, out.at[g])

    pltpu.make_async_copy(
        weight.at[idxw_v.at[SPW - 1, pl.ds(0, ROWS_PER_GATHER)]], rows0, sem0).wait()
    pltpu.make_async_copy(
        weight.at[idxw_v.at[SPW - 1, pl.ds(ROWS_PER_GATHER, ROWS_PER_GATHER)]], rows1, sem1).wait()


@jax.jit
def _run(w_indices, w_values, b_indices, b_values, weight, bias, us, them, ftv):
    mesh = plsc.VectorSubcoreMesh(core_axis_name="c", subcore_axis_name="s")
    f = pl.kernel(
        _sc_kernel,
        out_type=jax.ShapeDtypeStruct((B, D), jnp.float32),
        mesh=mesh,
        scratch_types=[
            pltpu.VMEM((SPW, A), jnp.int32),
            pltpu.VMEM((SPW, A), jnp.int32),
            pltpu.VMEM((SPW, A), jnp.float32),
            pltpu.VMEM((SPW, A), jnp.float32),
            pltpu.VMEM((SPW,), jnp.float32),
            pltpu.VMEM((SPW,), jnp.float32),
            pltpu.VMEM((16,), jnp.float32),
            pltpu.VMEM((D,), jnp.float32),
            pltpu.VMEM((ROWS_PER_GATHER, D), jnp.float32),
            pltpu.VMEM((ROWS_PER_GATHER, D), jnp.float32),
            pltpu.VMEM((ACC,), jnp.float32),
            pltpu.VMEM((ACC,), jnp.float32),
            pltpu.VMEM((D,), jnp.float32),
            pltpu.SemaphoreType.DMA,
            pltpu.SemaphoreType.DMA,
        ],
        compiler_params=pltpu.CompilerParams(use_tc_tiling_on_sc=False,
                                             needs_layout_passes=False),
    )
    return f(w_indices, w_values, b_indices, b_values, us, them, weight, bias, ftv)


def kernel(w_indices, w_values, b_indices, b_values, weight, bias, us, them, ft_max_val):
    ftv = jnp.broadcast_to(jnp.asarray(ft_max_val, jnp.float32), (16,))
    return _run(w_indices.astype(jnp.int32), w_values, b_indices.astype(jnp.int32),
                b_values, weight, bias, us.reshape(B), them.reshape(B), ftv)


# speedup vs baseline: 2.9408x; 2.9408x over previous
import functools

import jax
import jax.numpy as jnp
from jax import lax
from jax.experimental import pallas as pl
from jax.experimental.pallas import tpu as pltpu
from jax.experimental.pallas import tpu_sc as plsc

L1 = 3072
NPSQT = 8
D = L1 + NPSQT
H = L1 // 2
B = 1024
A = 32
NC = 2
NS = 16
NW = NC * NS
SPW = B // NW
CH = L1 // 16
ROWS_PER_GATHER = 16
ACC = L1 + 16


def _sc_kernel(w_idx, w_val, b_idx, b_val, us, them, weight, bias, ftv, out,
               idxw_v, idxb_v, vw_v, vb_v, us_v, them_v, ft_v, bias_v,
               rows0, rows1, acc_w, acc_b, out_stage, sem0, sem1):
    wid = lax.axis_index("s") * NC + lax.axis_index("c")
    base = wid * SPW

    pltpu.sync_copy(w_idx.at[pl.ds(base, SPW)], idxw_v)
    pltpu.sync_copy(b_idx.at[pl.ds(base, SPW)], idxb_v)
    pltpu.sync_copy(w_val.at[pl.ds(base, SPW)], vw_v)
    pltpu.sync_copy(b_val.at[pl.ds(base, SPW)], vb_v)
    pltpu.sync_copy(us.at[pl.ds(base, SPW)], us_v)
    pltpu.sync_copy(them.at[pl.ds(base, SPW)], them_v)
    pltpu.sync_copy(bias, bias_v)
    pltpu.sync_copy(ftv, ft_v)

    lane = lax.broadcasted_iota(jnp.int32, (16,), 0)
    tail_mask = (lane >= 8).astype(jnp.float32)
    rows = (rows0, rows1)
    sems = (sem0, sem1)
    idxs = (idxw_v, idxb_v)
    vals = (vw_v, vb_v)

    def start_gather(s, p, h):
        src = weight.at[idxs[p].at[s, pl.ds(h * ROWS_PER_GATHER, ROWS_PER_GATHER)]]
        return pltpu.async_copy(src, rows[h], sems[h])

    def full16(v):
        return jnp.full((16,), v, dtype=jnp.int32)

    G = 8
    NG = CH // G

    def mac(s, p, h):
        acc = (acc_w, acc_b)[p]
        rbuf = rows[h]
        vref = vals[p]
        first = (h == 0)

        vjs = [plsc.load_gather(vref, [full16(s), full16(h * ROWS_PER_GATHER + j)])
               for j in range(ROWS_PER_GATHER)]

        @pl.loop(0, NG)
        def _group(gi):
            goff = gi * (G * 16)
            if first:
                accs = [bias_v[pl.ds(goff + k * 16, 16)] for k in range(G)]
            else:
                accs = [acc[pl.ds(goff + k * 16, 16)] for k in range(G)]
            for j in range(ROWS_PER_GATHER):
                for k in range(G):
                    accs[k] = accs[k] + rbuf[j, pl.ds(goff + k * 16, 16)] * vjs[j]
            for k in range(G):
                acc[pl.ds(goff + k * 16, 16)] = accs[k]

        tacc = bias_v[pl.ds(D - 16, 16)] if first else acc[pl.ds(L1, 16)]
        tsum = None
        for j in range(ROWS_PER_GATHER):
            t = plsc.load_gather(rbuf, [full16(j), lane + (D - 16)]) * vjs[j]
            tsum = t if tsum is None else tsum + t
        acc[pl.ds(L1, 16)] = tacc + tsum * tail_mask if not first else (
            tacc * tail_mask + tsum * tail_mask)

    start_gather(0, 0, 0)
    start_gather(0, 0, 1)

    @pl.loop(0, SPW)
    def _sample(s):
        g = base + s
        s_next = jnp.minimum(s + 1, SPW - 1)

        pltpu.make_async_copy(
            weight.at[idxw_v.at[s, pl.ds(0, ROWS_PER_GATHER)]], rows0, sem0).wait()
        mac(s, 0, 0)
        d_b0 = start_gather(s, 1, 0)

        pltpu.make_async_copy(
            weight.at[idxw_v.at[s, pl.ds(ROWS_PER_GATHER, ROWS_PER_GATHER)]], rows1, sem1).wait()
        mac(s, 0, 1)
        d_b1 = start_gather(s, 1, 1)

        d_b0.wait()
        mac(s, 1, 0)
        start_gather(s_next, 0, 0)

        d_b1.wait()
        mac(s, 1, 1)
        start_gather(s_next, 0, 1)

        usv = plsc.load_gather(us_v, [full16(s)])
        thv = plsc.load_gather(them_v, [full16(s)])
        ftm = ft_v[...]
        inv = 1.0 / ftm
        zero = jnp.zeros((16,), jnp.float32)

        def clampf(x):
            return jnp.minimum(jnp.maximum(x, zero), ftm)

        @pl.loop(0, H // 16, unroll=2)
        def _fin(c):
            o1 = c * 16
            o2 = H + c * 16
            aw1 = acc_w[pl.ds(o1, 16)]
            aw2 = acc_w[pl.ds(o2, 16)]
            ab1 = acc_b[pl.ds(o1, 16)]
            ab2 = acc_b[pl.ds(o2, 16)]
            w1 = clampf(usv * aw1 + thv * ab1)
            w2 = clampf(usv * aw2 + thv * ab2)
            v1 = clampf(usv * ab1 + thv * aw1)
            v2 = clampf(usv * ab2 + thv * aw2)
            out_stage[pl.ds(o1, 16)] = w1 * w2 * inv
            out_stage[pl.ds(H + o1, 16)] = v1 * v2 * inv

        pq = (acc_w[pl.ds(L1, 16)] - acc_b[pl.ds(L1, 16)]) * (usv - 0.5)
        plsc.store_scatter(out_stage, [lane + (D - 16)], pq, mask=lane >= 8)

        pltpu.sync_copy(out_stage, out.at[g])

    pltpu.make_async_copy(
        weight.at[idxw_v.at[SPW - 1, pl.ds(0, ROWS_PER_GATHER)]], rows0, sem0).wait()
    pltpu.make_async_copy(
        weight.at[idxw_v.at[SPW - 1, pl.ds(ROWS_PER_GATHER, ROWS_PER_GATHER)]], rows1, sem1).wait()


@jax.jit
def _run(w_indices, w_values, b_indices, b_values, weight, bias, us, them, ftv):
    mesh = plsc.VectorSubcoreMesh(core_axis_name="c", subcore_axis_name="s")
    f = pl.kernel(
        _sc_kernel,
        out_type=jax.ShapeDtypeStruct((B, D), jnp.float32),
        mesh=mesh,
        scratch_types=[
            pltpu.VMEM((SPW, A), jnp.int32),
            pltpu.VMEM((SPW, A), jnp.int32),
            pltpu.VMEM((SPW, A), jnp.float32),
            pltpu.VMEM((SPW, A), jnp.float32),
            pltpu.VMEM((SPW,), jnp.float32),
            pltpu.VMEM((SPW,), jnp.float32),
            pltpu.VMEM((16,), jnp.float32),
            pltpu.VMEM((D,), jnp.float32),
            pltpu.VMEM((ROWS_PER_GATHER, D), jnp.float32),
            pltpu.VMEM((ROWS_PER_GATHER, D), jnp.float32),
            pltpu.VMEM((ACC,), jnp.float32),
            pltpu.VMEM((ACC,), jnp.float32),
            pltpu.VMEM((D,), jnp.float32),
            pltpu.SemaphoreType.DMA,
            pltpu.SemaphoreType.DMA,
        ],
        compiler_params=pltpu.CompilerParams(use_tc_tiling_on_sc=False,
                                             needs_layout_passes=False),
    )
    return f(w_indices, w_values, b_indices, b_values, us, them, weight, bias, ftv)


def kernel(w_indices, w_values, b_indices, b_values, weight, bias, us, them, ft_max_val):
    ftv = jnp.broadcast_to(jnp.asarray(ft_max_val, jnp.float32), (16,))
    return _run(w_indices.astype(jnp.int32), w_values, b_indices.astype(jnp.int32),
                b_values, weight, bias, us.reshape(B), them.reshape(B), ftv)
